# baseline (speedup 1.0000x reference)
import functools

import jax
import jax.numpy as jnp
from jax import lax
from jax.experimental import pallas as pl
from jax.experimental.pallas import tpu as pltpu
from jax.experimental.pallas import tpu_sc as plsc

B, N, D, E, K, H, FF, OUT = 2, 2048, 1024, 8, 2, 128, 1024, 4
T = B * N
TB = 1024
NTB = T // TB
TILE = 128
PADTOT = 2 * T + E * TILE
NT = PADTOT // TILE
NW = 32
CHUNK = T // NW


D2 = D // 2


def _router_body(x_ref, nr_ref, w1_ref, b1_ref, w2_ref, b2_ref, rb_ref,
                 e0_ref, e1_ref, r0_ref, r1_ref, g0_ref, g1_ref,
                 off_ref, meta_ref, aux_ref, xp_ref, carry_cnt, carry_gm):
    i = pl.program_id(0)

    @pl.when(i == 0)
    def _():
        carry_cnt[...] = jnp.zeros((1, E), jnp.float32)
        carry_gm[...] = jnp.zeros((1, E), jnp.float32)

    x = x_ref[...]
    u = lax.bitcast_convert_type(x, jnp.uint32)
    rne = lambda v: (v + jnp.uint32(0x7FFF)
                     + ((v >> jnp.uint32(16)) & jnp.uint32(1))) >> jnp.uint32(16)
    ra = rne(u[:, :D2])
    rb = rne(u[:, D2:])
    xp_ref[...] = lax.bitcast_convert_type(ra | (rb << jnp.uint32(16)),
                                           jnp.int32)

    h = jnp.maximum(jnp.dot(x, w1_ref[...], preferred_element_type=jnp.float32)
                    + b1_ref[...], 0.0)
    logits = (jnp.dot(h, w2_ref[...], preferred_element_type=jnp.float32)
              + b2_ref[...] + rb_ref[...] * nr_ref[...])

    iota_e = lax.broadcasted_iota(jnp.int32, (TB, E), 1)
    m0 = jnp.max(logits, axis=1, keepdims=True)
    i0 = jnp.min(jnp.where(logits == m0, iota_e, E), axis=1, keepdims=True)
    masked = jnp.where(iota_e == i0, -1e30, logits)
    m1 = jnp.max(masked, axis=1, keepdims=True)
    i1 = jnp.min(jnp.where(masked == m1, iota_e, E), axis=1, keepdims=True)

    e1x = jnp.exp(m1 - m0)
    g0 = 1.0 / (1.0 + e1x)
    g1 = e1x * g0

    oh0 = (iota_e == i0).astype(jnp.float32)
    oh1 = (iota_e == i1).astype(jnp.float32)
    s = oh0 + oh1

    ir = lax.broadcasted_iota(jnp.int32, (TB, TB), 0)
    ic = lax.broadcasted_iota(jnp.int32, (TB, TB), 1)
    ltri = (ic < ir).astype(jnp.float32)
    excl = jnp.dot(ltri, s, preferred_element_type=jnp.float32)
    base = excl + carry_cnt[...]

    r0 = jnp.sum(oh0 * base, axis=1, keepdims=True)
    r1 = jnp.sum(oh1 * base, axis=1, keepdims=True)

    carry_cnt[...] = carry_cnt[...] + jnp.sum(s, axis=0, keepdims=True)
    carry_gm[...] = carry_gm[...] + jnp.sum(g0 * oh0 + g1 * oh1, axis=0,
                                            keepdims=True)

    e0_ref[...] = i0
    e1_ref[...] = i1
    r0_ref[...] = r0.astype(jnp.int32)
    r1_ref[...] = r1.astype(jnp.int32)
    g0_ref[...] = g0
    g1_ref[...] = g1

    @pl.when(i == NTB - 1)
    def _():
        cnt = carry_cnt[...]
        width = jnp.ceil(cnt / TILE) * TILE
        iu = lax.broadcasted_iota(jnp.int32, (E, E), 0)
        ju = lax.broadcasted_iota(jnp.int32, (E, E), 1)
        sut = (iu < ju).astype(jnp.float32)
        off = jnp.dot(width, sut, preferred_element_type=jnp.float32)
        off_ref[...] = off.astype(jnp.int32)
        lane = lax.broadcasted_iota(jnp.int32, (8, 128), 1)
        tstart = (lane * TILE).astype(jnp.float32)
        te = jnp.zeros((8, 128), jnp.int32)
        for e in range(1, E):
            te = te + (tstart >= off[:, e:e + 1]).astype(jnp.int32)
        nreal = ((off[:, E - 1:E] + width[:, E - 1:E]) / TILE).astype(jnp.int32)
        row = lax.broadcasted_iota(jnp.int32, (8, 128), 0)
        meta_ref[...] = jnp.where(row == 0, te, nreal)
        scale = 0.01 * E / (float(T) * float(T))
        aux_ref[...] = jnp.sum(cnt * carry_gm[...],
                               keepdims=True).reshape(1, 1) * scale


def _router_call(x2, nr2, Wr1, br1, Wr2, br2, rbias):
    out_shapes = (
        jax.ShapeDtypeStruct((T, 1), jnp.int32),
        jax.ShapeDtypeStruct((T, 1), jnp.int32),
        jax.ShapeDtypeStruct((T, 1), jnp.int32),
        jax.ShapeDtypeStruct((T, 1), jnp.int32),
        jax.ShapeDtypeStruct((T, 1), jnp.float32),
        jax.ShapeDtypeStruct((T, 1), jnp.float32),
        jax.ShapeDtypeStruct((1, E), jnp.int32),
        jax.ShapeDtypeStruct((8, 128), jnp.int32),
        jax.ShapeDtypeStruct((1, 1), jnp.float32),
        jax.ShapeDtypeStruct((T, D2), jnp.int32),
    )
    tok_vec = pl.BlockSpec((TB, 1), lambda i: (i, 0))
    return pl.pallas_call(
        _router_body,
        grid=(NTB,),
        in_specs=[
            pl.BlockSpec((TB, D), lambda i: (i, 0)),
            pl.BlockSpec((TB, E), lambda i: (i, 0)),
            pl.BlockSpec((D, H), lambda i: (0, 0)),
            pl.BlockSpec((1, H), lambda i: (0, 0)),
            pl.BlockSpec((H, E), lambda i: (0, 0)),
            pl.BlockSpec((1, E), lambda i: (0, 0)),
            pl.BlockSpec((1, E), lambda i: (0, 0)),
        ],
        out_specs=(tok_vec, tok_vec, tok_vec, tok_vec, tok_vec, tok_vec,
                   pl.BlockSpec((1, E), lambda i: (0, 0)),
                   pl.BlockSpec((8, 128), lambda i: (0, 0)),
                   pl.BlockSpec((1, 1), lambda i: (0, 0)),
                   pl.BlockSpec((TB, D2), lambda i: (i, 0))),
        out_shape=out_shapes,
        scratch_shapes=[pltpu.VMEM((1, E), jnp.float32),
                        pltpu.VMEM((1, E), jnp.float32)],
    )(x2, nr2, Wr1, br1.reshape(1, H), Wr2, br2.reshape(1, E),
      rbias.reshape(1, E))


def _make_dispatch():
    mesh = plsc.VectorSubcoreMesh(core_axis_name="c", subcore_axis_name="s")

    @functools.partial(
        pl.kernel, mesh=mesh,
        out_type=jax.ShapeDtypeStruct((PADTOT, D // 2), jnp.int32),
        compiler_params=pltpu.CompilerParams(needs_layout_passes=False),
        scratch_types=[
            pltpu.VMEM((CHUNK, D // 2), jnp.int32),
            pltpu.VMEM((E,), jnp.int32),
            pltpu.VMEM((CHUNK,), jnp.int32),
            pltpu.VMEM((CHUNK,), jnp.int32),
            pltpu.VMEM((CHUNK,), jnp.int32),
            pltpu.VMEM((CHUNK,), jnp.int32),
            pltpu.VMEM((CHUNK,), jnp.int32),
            pltpu.VMEM((CHUNK,), jnp.int32),
            pltpu.SemaphoreType.DMA,
            pltpu.SemaphoreType.DMA,
        ],
    )
    def dispatch(x_hbm, e0_hbm, e1_hbm, r0_hbm, r1_hbm, off_hbm, xg_hbm,
                 buf, offv, e0v, e1v, r0v, r1v, i0v, i1v, s0, s1):
        wid = lax.axis_index("s") * 2 + lax.axis_index("c")
        base = wid * CHUNK
        cx = pltpu.async_copy(x_hbm.at[pl.ds(base, CHUNK)], buf, s0)
        pltpu.sync_copy(off_hbm, offv)
        pltpu.sync_copy(e0_hbm.at[pl.ds(base, CHUNK)], e0v)
        pltpu.sync_copy(e1_hbm.at[pl.ds(base, CHUNK)], e1v)
        pltpu.sync_copy(r0_hbm.at[pl.ds(base, CHUNK)], r0v)
        pltpu.sync_copy(r1_hbm.at[pl.ds(base, CHUNK)], r1v)
        for v in range(CHUNK // 16):
            sl = pl.ds(v * 16, 16)
            i0v[sl] = plsc.load_gather(offv, [e0v[sl]]) + r0v[sl]
            i1v[sl] = plsc.load_gather(offv, [e1v[sl]]) + r1v[sl]
        cx.wait()
        c0 = pltpu.async_copy(buf, xg_hbm.at[i0v], s0)
        c1 = pltpu.async_copy(buf, xg_hbm.at[i1v], s1)
        c0.wait()
        c1.wait()

    return dispatch


def _ffn_body(te_ref, nr_ref, xg_ref, w1_ref, b1_ref, w2_ref, b2_ref,
              w3_ref, b3_ref, y_ref, w1b, w2b, w3b):
    i = pl.program_id(0)

    @pl.when(i < nr_ref[0])
    def _():
        prev = te_ref[jnp.maximum(i - 1, 0)]
        @pl.when((i == 0) | (te_ref[i] != prev))
        def _():
            w1b[...] = w1_ref[0].astype(jnp.bfloat16)
            w2b[...] = w2_ref[0].astype(jnp.bfloat16)
            w3b[...] = w3_ref[0].astype(jnp.bfloat16)
        w = xg_ref[...]
        xa = lax.bitcast_convert_type(w << 16, jnp.float32).astype(jnp.bfloat16)
        xb = lax.bitcast_convert_type(w & jnp.int32(-65536),
                                      jnp.float32).astype(jnp.bfloat16)
        h1 = jnp.maximum(
            jnp.dot(xa, w1b[:D2], preferred_element_type=jnp.float32)
            + jnp.dot(xb, w1b[D2:], preferred_element_type=jnp.float32)
            + b1_ref[0], 0.0).astype(jnp.bfloat16)
        h2 = jnp.maximum(
            jnp.dot(h1, w2b[...], preferred_element_type=jnp.float32)
            + b2_ref[0], 0.0).astype(jnp.bfloat16)
        y_ref[...] = (jnp.dot(h2, w3b[...], preferred_element_type=jnp.float32)
                      + b3_ref[0])


def _ffn_call(te, nreal, xg, w1, b1, w2, b2, w3, b3):
    spec = pltpu.PrefetchScalarGridSpec(
        num_scalar_prefetch=2,
        grid=(NT,),
        in_specs=[
            pl.BlockSpec((TILE, D2), lambda i, te, nr: (i, 0)),
            pl.BlockSpec((1, D, FF), lambda i, te, nr: (te[i], 0, 0)),
            pl.BlockSpec((1, 1, FF), lambda i, te, nr: (te[i], 0, 0)),
            pl.BlockSpec((1, FF, FF), lambda i, te, nr: (te[i], 0, 0)),
            pl.BlockSpec((1, 1, FF), lambda i, te, nr: (te[i], 0, 0)),
            pl.BlockSpec((1, FF, OUT), lambda i, te, nr: (te[i], 0, 0)),
            pl.BlockSpec((1, 1, OUT), lambda i, te, nr: (te[i], 0, 0)),
        ],
        out_specs=pl.BlockSpec((TILE, OUT), lambda i, te, nr: (i, 0)),
        scratch_shapes=[pltpu.VMEM((D, FF), jnp.bfloat16),
                        pltpu.VMEM((FF, FF), jnp.bfloat16),
                        pltpu.VMEM((FF, OUT), jnp.bfloat16)],
    )
    return pl.pallas_call(
        _ffn_body, grid_spec=spec,
        out_shape=jax.ShapeDtypeStruct((PADTOT, OUT), jnp.float32),
    )(te, nreal, xg, w1, b1.reshape(E, 1, FF), w2, b2.reshape(E, 1, FF),
      w3, b3.reshape(E, 1, OUT))


def _make_combine():
    mesh = plsc.VectorSubcoreMesh(core_axis_name="c", subcore_axis_name="s")

    @functools.partial(
        pl.kernel, mesh=mesh,
        out_type=jax.ShapeDtypeStruct((T * OUT,), jnp.float32),
        compiler_params=pltpu.CompilerParams(needs_layout_passes=False),
        scratch_types=[
            pltpu.VMEM((PADTOT * OUT,), jnp.float32),
            pltpu.VMEM((E,), jnp.int32),
            pltpu.VMEM((CHUNK,), jnp.int32),
            pltpu.VMEM((CHUNK,), jnp.int32),
            pltpu.VMEM((CHUNK,), jnp.int32),
            pltpu.VMEM((CHUNK,), jnp.int32),
            pltpu.VMEM((CHUNK,), jnp.float32),
            pltpu.VMEM((CHUNK,), jnp.float32),
            pltpu.VMEM((CHUNK * OUT,), jnp.float32),
        ],
    )
    def combine(y_hbm, e0_hbm, e1_hbm, r0_hbm, r1_hbm, off_hbm,
                g0_hbm, g1_hbm, out_hbm,
                ytile, offv, e0v, e1v, r0v, r1v, g0v, g1v, ob):
        wid = lax.axis_index("s") * 2 + lax.axis_index("c")
        base = wid * CHUNK
        pltpu.sync_copy(y_hbm, ytile)
        pltpu.sync_copy(off_hbm, offv)
        pltpu.sync_copy(e0_hbm.at[pl.ds(base, CHUNK)], e0v)
        pltpu.sync_copy(e1_hbm.at[pl.ds(base, CHUNK)], e1v)
        pltpu.sync_copy(r0_hbm.at[pl.ds(base, CHUNK)], r0v)
        pltpu.sync_copy(r1_hbm.at[pl.ds(base, CHUNK)], r1v)
        pltpu.sync_copy(g0_hbm.at[pl.ds(base, CHUNK)], g0v)
        pltpu.sync_copy(g1_hbm.at[pl.ds(base, CHUNK)], g1v)
        lanes = lax.broadcasted_iota(jnp.int32, (16,), 0)
        for v in range(CHUNK // 16):
            sl = pl.ds(v * 16, 16)
            rows0 = (plsc.load_gather(offv, [e0v[sl]]) + r0v[sl]) * OUT
            rows1 = (plsc.load_gather(offv, [e1v[sl]]) + r1v[sl]) * OUT
            ga = g0v[sl]
            gb = g1v[sl]
            orow = (lanes + v * 16) * OUT
            for c in range(OUT):
                ya = plsc.load_gather(ytile, [rows0 + c])
                yb = plsc.load_gather(ytile, [rows1 + c])
                plsc.store_scatter(ob, [orow + c], ga * ya + gb * yb)
        pltpu.sync_copy(ob, out_hbm.at[pl.ds(base * OUT, CHUNK * OUT)])

    return combine


def kernel(x, node_regions, Wr1, br1, Wr2, br2, rbias,
           We1, be1, We2, be2, We3, be3):
    x2 = x.reshape(T, D)
    nr2 = node_regions.reshape(T, E)

    e0, e1, r0, r1, g0, g1, off, meta, aux, xp = _router_call(
        x2, nr2, Wr1, br1, Wr2, br2, rbias)

    e0f, e1f = e0.reshape(T), e1.reshape(T)
    r0f, r1f = r0.reshape(T), r1.reshape(T)
    offf = off.reshape(E)
    xg = _make_dispatch()(xp, e0f, e1f, r0f, r1f, offf)
    y = _ffn_call(meta[0], meta[1, :1], xg, We1, be1, We2, be2, We3, be3)
    out = _make_combine()(y.reshape(PADTOT * OUT), e0f, e1f, r0f, r1f, offf,
                          g0.reshape(T), g1.reshape(T))
    return (out.reshape(B, N, OUT), aux[0, 0])


# speedup vs baseline: 1.1473x; 1.1473x over previous
import functools

import jax
import jax.numpy as jnp
from jax import lax
from jax.experimental import pallas as pl
from jax.experimental.pallas import tpu as pltpu
from jax.experimental.pallas import tpu_sc as plsc

B, N, D, E, K, H, FF, OUT = 2, 2048, 1024, 8, 2, 128, 1024, 4
T = B * N
TB = 1024
NTB = T // TB
TILE = 512
PADTOT = 2 * T + E * TILE
NT = PADTOT // TILE
NW = 32
CHUNK = T // NW


D2 = D // 2


def _router_body(x_ref, nr_ref, w1_ref, b1_ref, w2_ref, b2_ref, rb_ref,
                 e0_ref, e1_ref, r0_ref, r1_ref, g0_ref, g1_ref,
                 off_ref, meta_ref, aux_ref, xp_ref, carry_cnt, carry_gm):
    i = pl.program_id(0)

    @pl.when(i == 0)
    def _():
        carry_cnt[...] = jnp.zeros((1, E), jnp.float32)
        carry_gm[...] = jnp.zeros((1, E), jnp.float32)

    x = x_ref[...]
    u = lax.bitcast_convert_type(x, jnp.uint32)
    rne = lambda v: (v + jnp.uint32(0x7FFF)
                     + ((v >> jnp.uint32(16)) & jnp.uint32(1))) >> jnp.uint32(16)
    ra = rne(u[:, :D2])
    rb = rne(u[:, D2:])
    xp_ref[...] = lax.bitcast_convert_type(ra | (rb << jnp.uint32(16)),
                                           jnp.int32)

    h = jnp.maximum(jnp.dot(x, w1_ref[...], preferred_element_type=jnp.float32)
                    + b1_ref[...], 0.0)
    logits = (jnp.dot(h, w2_ref[...], preferred_element_type=jnp.float32)
              + b2_ref[...] + rb_ref[...] * nr_ref[...])

    iota_e = lax.broadcasted_iota(jnp.int32, (TB, E), 1)
    m0 = jnp.max(logits, axis=1, keepdims=True)
    i0 = jnp.min(jnp.where(logits == m0, iota_e, E), axis=1, keepdims=True)
    masked = jnp.where(iota_e == i0, -1e30, logits)
    m1 = jnp.max(masked, axis=1, keepdims=True)
    i1 = jnp.min(jnp.where(masked == m1, iota_e, E), axis=1, keepdims=True)

    e1x = jnp.exp(m1 - m0)
    g0 = 1.0 / (1.0 + e1x)
    g1 = e1x * g0

    oh0 = (iota_e == i0).astype(jnp.float32)
    oh1 = (iota_e == i1).astype(jnp.float32)
    s = oh0 + oh1

    ir = lax.broadcasted_iota(jnp.int32, (TB, TB), 0)
    ic = lax.broadcasted_iota(jnp.int32, (TB, TB), 1)
    ltri = (ic < ir).astype(jnp.float32)
    excl = jnp.dot(ltri, s, preferred_element_type=jnp.float32)
    base = excl + carry_cnt[...]

    r0 = jnp.sum(oh0 * base, axis=1, keepdims=True)
    r1 = jnp.sum(oh1 * base, axis=1, keepdims=True)

    carry_cnt[...] = carry_cnt[...] + jnp.sum(s, axis=0, keepdims=True)
    carry_gm[...] = carry_gm[...] + jnp.sum(g0 * oh0 + g1 * oh1, axis=0,
                                            keepdims=True)

    e0_ref[...] = i0
    e1_ref[...] = i1
    r0_ref[...] = r0.astype(jnp.int32)
    r1_ref[...] = r1.astype(jnp.int32)
    g0_ref[...] = g0
    g1_ref[...] = g1

    @pl.when(i == NTB - 1)
    def _():
        cnt = carry_cnt[...]
        width = jnp.ceil(cnt / TILE) * TILE
        iu = lax.broadcasted_iota(jnp.int32, (E, E), 0)
        ju = lax.broadcasted_iota(jnp.int32, (E, E), 1)
        sut = (iu < ju).astype(jnp.float32)
        off = jnp.dot(width, sut, preferred_element_type=jnp.float32)
        off_ref[...] = off.astype(jnp.int32)
        lane = lax.broadcasted_iota(jnp.int32, (8, 128), 1)
        tstart = (lane * TILE).astype(jnp.float32)
        te = jnp.zeros((8, 128), jnp.int32)
        for e in range(1, E):
            te = te + (tstart >= off[:, e:e + 1]).astype(jnp.int32)
        nreal = ((off[:, E - 1:E] + width[:, E - 1:E]) / TILE).astype(jnp.int32)
        row = lax.broadcasted_iota(jnp.int32, (8, 128), 0)
        meta_ref[...] = jnp.where(row == 0, te, nreal)
        scale = 0.01 * E / (float(T) * float(T))
        aux_ref[...] = jnp.sum(cnt * carry_gm[...],
                               keepdims=True).reshape(1, 1) * scale


def _router_call(x2, nr2, Wr1, br1, Wr2, br2, rbias):
    out_shapes = (
        jax.ShapeDtypeStruct((T, 1), jnp.int32),
        jax.ShapeDtypeStruct((T, 1), jnp.int32),
        jax.ShapeDtypeStruct((T, 1), jnp.int32),
        jax.ShapeDtypeStruct((T, 1), jnp.int32),
        jax.ShapeDtypeStruct((T, 1), jnp.float32),
        jax.ShapeDtypeStruct((T, 1), jnp.float32),
        jax.ShapeDtypeStruct((1, E), jnp.int32),
        jax.ShapeDtypeStruct((8, 128), jnp.int32),
        jax.ShapeDtypeStruct((1, 1), jnp.float32),
        jax.ShapeDtypeStruct((T, D2), jnp.int32),
    )
    tok_vec = pl.BlockSpec((TB, 1), lambda i: (i, 0))
    return pl.pallas_call(
        _router_body,
        grid=(NTB,),
        in_specs=[
            pl.BlockSpec((TB, D), lambda i: (i, 0)),
            pl.BlockSpec((TB, E), lambda i: (i, 0)),
            pl.BlockSpec((D, H), lambda i: (0, 0)),
            pl.BlockSpec((1, H), lambda i: (0, 0)),
            pl.BlockSpec((H, E), lambda i: (0, 0)),
            pl.BlockSpec((1, E), lambda i: (0, 0)),
            pl.BlockSpec((1, E), lambda i: (0, 0)),
        ],
        out_specs=(tok_vec, tok_vec, tok_vec, tok_vec, tok_vec, tok_vec,
                   pl.BlockSpec((1, E), lambda i: (0, 0)),
                   pl.BlockSpec((8, 128), lambda i: (0, 0)),
                   pl.BlockSpec((1, 1), lambda i: (0, 0)),
                   pl.BlockSpec((TB, D2), lambda i: (i, 0))),
        out_shape=out_shapes,
        scratch_shapes=[pltpu.VMEM((1, E), jnp.float32),
                        pltpu.VMEM((1, E), jnp.float32)],
    )(x2, nr2, Wr1, br1.reshape(1, H), Wr2, br2.reshape(1, E),
      rbias.reshape(1, E))


def _make_dispatch():
    mesh = plsc.VectorSubcoreMesh(core_axis_name="c", subcore_axis_name="s")

    @functools.partial(
        pl.kernel, mesh=mesh,
        out_type=jax.ShapeDtypeStruct((PADTOT, D // 2), jnp.int32),
        compiler_params=pltpu.CompilerParams(needs_layout_passes=False),
        scratch_types=[
            pltpu.VMEM((CHUNK, D // 2), jnp.int32),
            pltpu.VMEM((E,), jnp.int32),
            pltpu.VMEM((CHUNK,), jnp.int32),
            pltpu.VMEM((CHUNK,), jnp.int32),
            pltpu.VMEM((CHUNK,), jnp.int32),
            pltpu.VMEM((CHUNK,), jnp.int32),
            pltpu.VMEM((CHUNK,), jnp.int32),
            pltpu.VMEM((CHUNK,), jnp.int32),
            pltpu.SemaphoreType.DMA,
            pltpu.SemaphoreType.DMA,
        ],
    )
    def dispatch(x_hbm, e0_hbm, e1_hbm, r0_hbm, r1_hbm, off_hbm, xg_hbm,
                 buf, offv, e0v, e1v, r0v, r1v, i0v, i1v, s0, s1):
        wid = lax.axis_index("s") * 2 + lax.axis_index("c")
        base = wid * CHUNK
        cx = pltpu.async_copy(x_hbm.at[pl.ds(base, CHUNK)], buf, s0)
        pltpu.sync_copy(off_hbm, offv)
        pltpu.sync_copy(e0_hbm.at[pl.ds(base, CHUNK)], e0v)
        pltpu.sync_copy(e1_hbm.at[pl.ds(base, CHUNK)], e1v)
        pltpu.sync_copy(r0_hbm.at[pl.ds(base, CHUNK)], r0v)
        pltpu.sync_copy(r1_hbm.at[pl.ds(base, CHUNK)], r1v)
        for v in range(CHUNK // 16):
            sl = pl.ds(v * 16, 16)
            i0v[sl] = plsc.load_gather(offv, [e0v[sl]]) + r0v[sl]
            i1v[sl] = plsc.load_gather(offv, [e1v[sl]]) + r1v[sl]
        cx.wait()
        c0 = pltpu.async_copy(buf, xg_hbm.at[i0v], s0)
        c1 = pltpu.async_copy(buf, xg_hbm.at[i1v], s1)
        c0.wait()
        c1.wait()

    return dispatch


def _ffn_body(te_ref, nr_ref, xg_ref, w1_ref, b1_ref, w2_ref, b2_ref,
              w3_ref, b3_ref, y_ref, w1b, w2b, w3b):
    i = pl.program_id(0)

    @pl.when(i < nr_ref[0])
    def _():
        prev = te_ref[jnp.maximum(i - 1, 0)]
        @pl.when((i == 0) | (te_ref[i] != prev))
        def _():
            w1b[...] = w1_ref[0].astype(jnp.bfloat16)
            w2b[...] = w2_ref[0].astype(jnp.bfloat16)
            w3b[...] = w3_ref[0].astype(jnp.bfloat16)
        w = xg_ref[...]
        xa = lax.bitcast_convert_type(w << 16, jnp.float32).astype(jnp.bfloat16)
        xb = lax.bitcast_convert_type(w & jnp.int32(-65536),
                                      jnp.float32).astype(jnp.bfloat16)
        h1 = jnp.maximum(
            jnp.dot(xa, w1b[:D2], preferred_element_type=jnp.float32)
            + jnp.dot(xb, w1b[D2:], preferred_element_type=jnp.float32)
            + b1_ref[0], 0.0).astype(jnp.bfloat16)
        h2 = jnp.maximum(
            jnp.dot(h1, w2b[...], preferred_element_type=jnp.float32)
            + b2_ref[0], 0.0).astype(jnp.bfloat16)
        y_ref[...] = (jnp.dot(h2, w3b[...], preferred_element_type=jnp.float32)
                      + b3_ref[0])


def _ffn_call(te, nreal, xg, w1, b1, w2, b2, w3, b3):
    spec = pltpu.PrefetchScalarGridSpec(
        num_scalar_prefetch=2,
        grid=(NT,),
        in_specs=[
            pl.BlockSpec((TILE, D2), lambda i, te, nr: (i, 0)),
            pl.BlockSpec((1, D, FF), lambda i, te, nr: (te[i], 0, 0)),
            pl.BlockSpec((1, 1, FF), lambda i, te, nr: (te[i], 0, 0)),
            pl.BlockSpec((1, FF, FF), lambda i, te, nr: (te[i], 0, 0)),
            pl.BlockSpec((1, 1, FF), lambda i, te, nr: (te[i], 0, 0)),
            pl.BlockSpec((1, FF, OUT), lambda i, te, nr: (te[i], 0, 0)),
            pl.BlockSpec((1, 1, OUT), lambda i, te, nr: (te[i], 0, 0)),
        ],
        out_specs=pl.BlockSpec((TILE, OUT), lambda i, te, nr: (i, 0)),
        scratch_shapes=[pltpu.VMEM((D, FF), jnp.bfloat16),
                        pltpu.VMEM((FF, FF), jnp.bfloat16),
                        pltpu.VMEM((FF, OUT), jnp.bfloat16)],
    )
    return pl.pallas_call(
        _ffn_body, grid_spec=spec,
        out_shape=jax.ShapeDtypeStruct((PADTOT, OUT), jnp.float32),
    )(te, nreal, xg, w1, b1.reshape(E, 1, FF), w2, b2.reshape(E, 1, FF),
      w3, b3.reshape(E, 1, OUT))


def _make_combine():
    mesh = plsc.VectorSubcoreMesh(core_axis_name="c", subcore_axis_name="s")

    @functools.partial(
        pl.kernel, mesh=mesh,
        out_type=jax.ShapeDtypeStruct((T * OUT,), jnp.float32),
        compiler_params=pltpu.CompilerParams(needs_layout_passes=False),
        scratch_types=[
            pltpu.VMEM((PADTOT * OUT,), jnp.float32),
            pltpu.VMEM((E,), jnp.int32),
            pltpu.VMEM((CHUNK,), jnp.int32),
            pltpu.VMEM((CHUNK,), jnp.int32),
            pltpu.VMEM((CHUNK,), jnp.int32),
            pltpu.VMEM((CHUNK,), jnp.int32),
            pltpu.VMEM((CHUNK,), jnp.float32),
            pltpu.VMEM((CHUNK,), jnp.float32),
            pltpu.VMEM((CHUNK * OUT,), jnp.float32),
        ],
    )
    def combine(y_hbm, e0_hbm, e1_hbm, r0_hbm, r1_hbm, off_hbm,
                g0_hbm, g1_hbm, out_hbm,
                ytile, offv, e0v, e1v, r0v, r1v, g0v, g1v, ob):
        wid = lax.axis_index("s") * 2 + lax.axis_index("c")
        base = wid * CHUNK
        pltpu.sync_copy(y_hbm, ytile)
        pltpu.sync_copy(off_hbm, offv)
        pltpu.sync_copy(e0_hbm.at[pl.ds(base, CHUNK)], e0v)
        pltpu.sync_copy(e1_hbm.at[pl.ds(base, CHUNK)], e1v)
        pltpu.sync_copy(r0_hbm.at[pl.ds(base, CHUNK)], r0v)
        pltpu.sync_copy(r1_hbm.at[pl.ds(base, CHUNK)], r1v)
        pltpu.sync_copy(g0_hbm.at[pl.ds(base, CHUNK)], g0v)
        pltpu.sync_copy(g1_hbm.at[pl.ds(base, CHUNK)], g1v)
        lanes = lax.broadcasted_iota(jnp.int32, (16,), 0)
        for v in range(CHUNK // 16):
            sl = pl.ds(v * 16, 16)
            rows0 = (plsc.load_gather(offv, [e0v[sl]]) + r0v[sl]) * OUT
            rows1 = (plsc.load_gather(offv, [e1v[sl]]) + r1v[sl]) * OUT
            ga = g0v[sl]
            gb = g1v[sl]
            orow = (lanes + v * 16) * OUT
            for c in range(OUT):
                ya = plsc.load_gather(ytile, [rows0 + c])
                yb = plsc.load_gather(ytile, [rows1 + c])
                plsc.store_scatter(ob, [orow + c], ga * ya + gb * yb)
        pltpu.sync_copy(ob, out_hbm.at[pl.ds(base * OUT, CHUNK * OUT)])

    return combine


def kernel(x, node_regions, Wr1, br1, Wr2, br2, rbias,
           We1, be1, We2, be2, We3, be3):
    x2 = x.reshape(T, D)
    nr2 = node_regions.reshape(T, E)

    e0, e1, r0, r1, g0, g1, off, meta, aux, xp = _router_call(
        x2, nr2, Wr1, br1, Wr2, br2, rbias)

    e0f, e1f = e0.reshape(T), e1.reshape(T)
    r0f, r1f = r0.reshape(T), r1.reshape(T)
    offf = off.reshape(E)
    xg = _make_dispatch()(xp, e0f, e1f, r0f, r1f, offf)
    y = _ffn_call(meta[0], meta[1, :1], xg, We1, be1, We2, be2, We3, be3)
    out = _make_combine()(y.reshape(PADTOT * OUT), e0f, e1f, r0f, r1f, offf,
                          g0.reshape(T), g1.reshape(T))
    return (out.reshape(B, N, OUT), aux[0, 0])
